# baseline (speedup 1.0000x reference)
import functools

import jax
import jax.numpy as jnp
from jax import lax
from jax.experimental import pallas as pl
from jax.experimental.pallas import tpu as pltpu
from jax.experimental.pallas import tpu_sc as plsc

B = 16384
N_CAT = 26
N_CONT = 16
CARD = 100000
DIM = 32
N_TOK = N_CAT + N_CONT
NROW = N_CAT * CARD
PACK = 128 // DIM
NWIDE = NROW // PACK

NC = 2
NS = 16
NW = NC * NS
BW = B // NW
GC = 128
NG = BW // GC
NST = N_CAT * NG
SUB = DIM // 8
LANES = 16

NTILE = NROW // 128
TAIL = NROW - NTILE * 128
CT = 2
NCHK = NTILE // CT
CPW = NCHK // NW
CEXTRA = NCHK - CPW * NW
NBANK = 4


def _splat(vec, i):
  return lax.gather(
      vec, jnp.full((LANES, 1), i, jnp.int32),
      dimension_numbers=lax.GatherDimensionNumbers(
          offset_dims=(), collapsed_slice_dims=(0,), start_index_map=(0,)),
      slice_sizes=(1,),
      mode=lax.GatherScatterMode.PROMISE_IN_BOUNDS)


def _tr_body(tt_hbm, wide_hbm, src_v, tr_v,
             lsem0, lsem1, lsem2, lsem3, osem0, osem1, osem2, osem3):
  c = lax.axis_index("c")
  s = lax.axis_index("s")
  wid = s * NC + c
  nc = CPW + jnp.where(wid < CEXTRA, 1, 0)
  c0 = wid * CPW + jnp.minimum(wid, CEXTRA)
  lsems = (lsem0, lsem1, lsem2, lsem3)
  osems = (osem0, osem1, osem2, osem3)
  iota16 = lax.iota(jnp.int32, LANES)

  def load_copy(t, b):
    return pltpu.make_async_copy(
        tt_hbm.at[:, pl.ds((c0 + t) * (CT * 128), CT * 128)],
        src_v.at[b], lsems[b])

  def out_copy(t, b):
    return pltpu.make_async_copy(
        tr_v.at[b],
        wide_hbm.at[pl.ds((c0 + t) * (CT * DIM), CT * DIM), :], osems[b])

  for b in range(NBANK - 1):
    @pl.when(b < nc)
    def _():
      load_copy(b, b).start()

  def body(to, carry):
    for b in range(NBANK):
      t = to * NBANK + b

      @pl.when(t < nc)
      def _():
        @pl.when(t + (NBANK - 1) < nc)
        def _():
          load_copy(t + (NBANK - 1), (b + NBANK - 1) % NBANK).start()

        @pl.when(t >= NBANK)
        def _():
          out_copy(t - NBANK, b).wait()
        load_copy(t, b).wait()
        for g in range((CT * 128) // LANES):
          r16 = iota16 + (g * LANES)
          row16 = lax.shift_right_logical(r16, 2)
          colb16 = (r16 & 3) * DIM

          def d_body(d, carry2):
            vals = src_v[b, d, pl.ds(g * LANES, LANES)]
            plsc.store_scatter(
                tr_v.at[b],
                [row16, colb16 + jnp.full((LANES,), d, jnp.int32)], vals)
            return carry2
          lax.fori_loop(0, DIM, d_body, 0)
        out_copy(t, b).start()
    return carry
  lax.fori_loop(0, (CPW + NBANK) // NBANK + 1, body, 0)

  for b in range(NBANK):
    out_copy(0, b).wait()


_tr_kernel = functools.partial(
    pl.kernel,
    mesh=plsc.VectorSubcoreMesh(core_axis_name="c", subcore_axis_name="s"),
    compiler_params=pltpu.CompilerParams(needs_layout_passes=False),
    out_type=jax.ShapeDtypeStruct((NWIDE, 128), jnp.float32),
    scratch_types=[
        pltpu.VMEM((NBANK, DIM, CT * 128), jnp.float32),
        pltpu.VMEM((NBANK, CT * DIM, 128), jnp.float32),
        pltpu.SemaphoreType.DMA,
        pltpu.SemaphoreType.DMA,
        pltpu.SemaphoreType.DMA,
        pltpu.SemaphoreType.DMA,
        pltpu.SemaphoreType.DMA,
        pltpu.SemaphoreType.DMA,
        pltpu.SemaphoreType.DMA,
        pltpu.SemaphoreType.DMA,
    ],
)(_tr_body)


def _sc_body(xcat_hbm, xcont_hbm, wide_hbm, w_hbm, bias_hbm, tail_hbm,
             out_hbm, idx_v, idxw_v, gat_v, tr_v, xc_v, w_v, bias_v,
             tail_v, cont_v, gsem0, gsem1, osem0, osem1, csem0, csem1):
  c = lax.axis_index("c")
  s = lax.axis_index("s")
  wid = s * NC + c
  base = wid * BW
  ctile0 = wid * NG
  gsems = (gsem0, gsem1)
  osems = (osem0, osem1)
  csems = (csem0, csem1)
  iota16 = lax.iota(jnp.int32, LANES)

  pltpu.sync_copy(w_hbm, w_v)
  pltpu.sync_copy(bias_hbm, bias_v)
  pltpu.sync_copy(tail_hbm, tail_v)
  pltpu.sync_copy(xcont_hbm.at[:, pl.ds(base, BW)], xc_v)
  pltpu.sync_copy(xcat_hbm.at[:, pl.ds(base, BW)], idx_v)

  def add_body(f, carry):
    off = f * CARD
    for k in range(BW // LANES):
      sl = pl.ds(k * LANES, LANES)
      v = idx_v[f, sl] + off
      idx_v[f, sl] = v
      idxw_v[f, sl] = lax.shift_right_logical(v, 2)
    return carry
  lax.fori_loop(0, N_CAT, add_body, 0)

  def gather_copy(st, p):
    f = st % N_CAT
    cg = st // N_CAT
    return pltpu.make_async_copy(
        wide_hbm.at[idxw_v.at[f, pl.ds(cg * GC, GC)]],
        gat_v.at[p], gsems[p])

  def out_copies(st, p):
    f = st % N_CAT
    cg = st // N_CAT
    return [
        pltpu.make_async_copy(
            tr_v.at[p, pl.ds(r * 8, 8), :],
            out_hbm.at[f, r, ctile0 + cg], osems[p])
        for r in range(SUB)
    ]

  def transpose_block(st, p):
    f = st % N_CAT
    cg = st // N_CAT

    def d_body(d, carry):
      dsplat = jnp.full((LANES,), d, jnp.int32)
      psplat = jnp.full((LANES,), p, jnp.int32)
      for g in range(GC // LANES):
        idx16 = idx_v[f, pl.ds(cg * GC + g * LANES, LANES)]
        col = (idx16 & 3) * DIM + dsplat
        vals = plsc.load_gather(
            gat_v, [psplat, iota16 + (g * LANES), col])
        tmask = idx16 >= (NROW - TAIL)
        tloc = jnp.maximum(idx16 - (NROW - TAIL), 0)
        tvals = plsc.load_gather(tail_v, [tloc, dsplat])
        tr_v[p, d, pl.ds(g * LANES, LANES)] = jnp.where(tmask, tvals, vals)
      return carry
    lax.fori_loop(0, DIM, d_body, 0)

  gather_copy(0, 0).start()

  def cat_body(sto, carry):
    for p in (0, 1):
      st = sto * 2 + p

      @pl.when(st + 1 < NST)
      def _():
        gather_copy(st + 1, 1 - p).start()

      @pl.when(st >= 2)
      def _():
        for cp in out_copies(st - 2, p):
          cp.wait()
      gather_copy(st, p).wait()
      transpose_block(st, p)
      for cp in out_copies(st, p):
        cp.start()
    return carry
  lax.fori_loop(0, NST // 2, cat_body, 0)

  for p in (0, 1):
    for cp in out_copies(NST - 2 + p, p):
      cp.wait()

  def cont_copies(st, p):
    j = st % N_CONT
    cg = st // N_CONT
    return [
        pltpu.make_async_copy(
            cont_v.at[p, pl.ds(r * 8, 8), :],
            out_hbm.at[N_CAT + j, r, ctile0 + cg], csems[p])
        for r in range(SUB)
    ]

  def cont_body(sto, carry):
    for p in (0, 1):
      st = sto * 2 + p
      j = st % N_CONT
      cg = st // N_CONT

      @pl.when(st >= 2)
      def _():
        for cp in cont_copies(st - 2, p):
          cp.wait()

      def d_body(d, carry2):
        dd = d // LANES
        wv = _splat(w_v[j, pl.ds(dd * LANES, LANES)], d % LANES)
        bv = _splat(bias_v[j, pl.ds(dd * LANES, LANES)], d % LANES)
        for k in range(GC // LANES):
          sl = pl.ds(k * LANES, LANES)
          cont_v[p, d, sl] = xc_v[j, pl.ds(cg * GC + k * LANES, LANES)] * wv + bv
        return carry2
      lax.fori_loop(0, DIM, d_body, 0)
      for cp in cont_copies(st, p):
        cp.start()
    return carry
  lax.fori_loop(0, (N_CONT * NG) // 2, cont_body, 0)

  for p in (0, 1):
    for cp in cont_copies(N_CONT * NG - 2 + p, p):
      cp.wait()


_sc_kernel = functools.partial(
    pl.kernel,
    mesh=plsc.VectorSubcoreMesh(core_axis_name="c", subcore_axis_name="s"),
    compiler_params=pltpu.CompilerParams(needs_layout_passes=False),
    out_type=jax.ShapeDtypeStruct((N_TOK, SUB, B // GC, 8, GC), jnp.float32),
    scratch_types=[
        pltpu.VMEM((N_CAT, BW), jnp.int32),
        pltpu.VMEM((N_CAT, BW), jnp.int32),
        pltpu.VMEM((2, GC, 128), jnp.float32),
        pltpu.VMEM((2, DIM, GC), jnp.float32),
        pltpu.VMEM((N_CONT, BW), jnp.float32),
        pltpu.VMEM((N_CONT, DIM), jnp.float32),
        pltpu.VMEM((N_CONT, DIM), jnp.float32),
        pltpu.VMEM((TAIL, DIM), jnp.float32),
        pltpu.VMEM((2, DIM, GC), jnp.float32),
        pltpu.SemaphoreType.DMA,
        pltpu.SemaphoreType.DMA,
        pltpu.SemaphoreType.DMA,
        pltpu.SemaphoreType.DMA,
        pltpu.SemaphoreType.DMA,
        pltpu.SemaphoreType.DMA,
    ],
)(_sc_body)


@jax.jit
def kernel(x_cat, x_cont, cat_table, cont_W, cont_b):
  xcat_t = jnp.transpose(x_cat.astype(jnp.int32))
  xcont_t = jnp.transpose(x_cont)
  tt = jnp.transpose(cat_table)
  wide = _tr_kernel(tt)
  tail = cat_table[NROW - TAIL:, :]
  out5 = _sc_kernel(xcat_t, xcont_t, wide, cont_W, cont_b, tail)
  out_t = jnp.transpose(out5, (0, 1, 3, 2, 4)).reshape(N_TOK, DIM, B)
  return jnp.transpose(out_t, (2, 0, 1))


# speedup vs baseline: 2.3417x; 2.3417x over previous
import functools

import jax
import jax.numpy as jnp
from jax import lax
from jax.experimental import pallas as pl
from jax.experimental.pallas import tpu as pltpu
from jax.experimental.pallas import tpu_sc as plsc

B = 16384
N_CAT = 26
N_CONT = 16
CARD = 100000
DIM = 32
N_TOK = N_CAT + N_CONT
NROW = N_CAT * CARD
PACK = 128 // DIM
NWIDE = NROW // PACK

NC = 2
NS = 16
NW = NC * NS
BW = B // NW
GC = 128
NG = BW // GC
NST = N_CAT * NG
SUB = DIM // 8
LANES = 16

NTILE = NROW // 128
TAIL = NROW - NTILE * 128
CT = 2
NCHK = NTILE // CT
CPW = NCHK // NW
CEXTRA = NCHK - CPW * NW
NBANK = 4


def _splat(vec, i):
  return lax.gather(
      vec, jnp.full((LANES, 1), i, jnp.int32),
      dimension_numbers=lax.GatherDimensionNumbers(
          offset_dims=(), collapsed_slice_dims=(0,), start_index_map=(0,)),
      slice_sizes=(1,),
      mode=lax.GatherScatterMode.PROMISE_IN_BOUNDS)


def _tr_body(tt_hbm, wide_hbm, src_v, tr_v,
             lsem0, lsem1, lsem2, lsem3, osem0, osem1, osem2, osem3):
  c = lax.axis_index("c")
  s = lax.axis_index("s")
  wid = s * NC + c
  nc = CPW + jnp.where(wid < CEXTRA, 1, 0)
  c0 = wid * CPW + jnp.minimum(wid, CEXTRA)
  lsems = (lsem0, lsem1, lsem2, lsem3)
  osems = (osem0, osem1, osem2, osem3)
  iota16 = lax.iota(jnp.int32, LANES)

  def load_copy(t, b):
    return pltpu.make_async_copy(
        tt_hbm.at[:, pl.ds((c0 + t) * (CT * 128), CT * 128)],
        src_v.at[b], lsems[b])

  def out_copy(t, b):
    return pltpu.make_async_copy(
        tr_v.at[b],
        wide_hbm.at[pl.ds((c0 + t) * (CT * DIM), CT * DIM), :], osems[b])

  for b in range(NBANK - 1):
    @pl.when(b < nc)
    def _():
      load_copy(b, b).start()

  def body(to, carry):
    for b in range(NBANK):
      t = to * NBANK + b

      @pl.when(t < nc)
      def _():
        @pl.when(t + (NBANK - 1) < nc)
        def _():
          load_copy(t + (NBANK - 1), (b + NBANK - 1) % NBANK).start()

        @pl.when(t >= NBANK)
        def _():
          out_copy(t - NBANK, b).wait()
        load_copy(t, b).wait()
        colb16 = (iota16 & 3) * DIM
        rowb16 = lax.shift_right_logical(iota16, 2)
        bsplat = jnp.full((LANES,), b, jnp.int32)

        def g_body(g, carry2):
          r16 = iota16 + g * LANES
          row16 = rowb16 + g * 4
          for k in range(LANES):
            dk16 = (iota16 + k) & (LANES - 1)
            for d0 in (0, LANES):
              d16 = dk16 + d0
              vals = plsc.load_gather(src_v, [bsplat, d16, r16])
              plsc.store_scatter(tr_v.at[b], [row16, colb16 + d16], vals)
          return carry2
        lax.fori_loop(0, (CT * 128) // LANES, g_body, 0)
        out_copy(t, b).start()
    return carry
  lax.fori_loop(0, (CPW + NBANK) // NBANK + 1, body, 0)

  for b in range(NBANK):
    out_copy(0, b).wait()


_tr_kernel = functools.partial(
    pl.kernel,
    mesh=plsc.VectorSubcoreMesh(core_axis_name="c", subcore_axis_name="s"),
    compiler_params=pltpu.CompilerParams(needs_layout_passes=False),
    out_type=jax.ShapeDtypeStruct((NWIDE, 128), jnp.float32),
    scratch_types=[
        pltpu.VMEM((NBANK, DIM, CT * 128), jnp.float32),
        pltpu.VMEM((NBANK, CT * DIM, 128), jnp.float32),
        pltpu.SemaphoreType.DMA,
        pltpu.SemaphoreType.DMA,
        pltpu.SemaphoreType.DMA,
        pltpu.SemaphoreType.DMA,
        pltpu.SemaphoreType.DMA,
        pltpu.SemaphoreType.DMA,
        pltpu.SemaphoreType.DMA,
        pltpu.SemaphoreType.DMA,
    ],
)(_tr_body)


def _sc_body(xcat_hbm, xcont_hbm, wide_hbm, w_hbm, bias_hbm, tail_hbm,
             out_hbm, idx_v, idxw_v, gat_v, tr_v, xc_v, w_v, bias_v,
             tail_v, cont_v, gsem0, gsem1, osem0, osem1, csem0, csem1):
  c = lax.axis_index("c")
  s = lax.axis_index("s")
  wid = s * NC + c
  base = wid * BW
  ctile0 = wid * NG
  gsems = (gsem0, gsem1)
  osems = (osem0, osem1)
  csems = (csem0, csem1)
  iota16 = lax.iota(jnp.int32, LANES)

  pltpu.sync_copy(w_hbm, w_v)
  pltpu.sync_copy(bias_hbm, bias_v)
  pltpu.sync_copy(tail_hbm, tail_v)
  pltpu.sync_copy(xcont_hbm.at[:, pl.ds(base, BW)], xc_v)
  pltpu.sync_copy(xcat_hbm.at[:, pl.ds(base, BW)], idx_v)

  def add_body(f, carry):
    off = f * CARD
    for k in range(BW // LANES):
      sl = pl.ds(k * LANES, LANES)
      v = idx_v[f, sl] + off
      idx_v[f, sl] = v
      idxw_v[f, sl] = lax.shift_right_logical(v, 2)
    return carry
  lax.fori_loop(0, N_CAT, add_body, 0)

  def gather_copy(st, p):
    f = st % N_CAT
    cg = st // N_CAT
    return pltpu.make_async_copy(
        wide_hbm.at[idxw_v.at[f, pl.ds(cg * GC, GC)]],
        gat_v.at[p], gsems[p])

  def out_copies(st, p):
    f = st % N_CAT
    cg = st // N_CAT
    return [
        pltpu.make_async_copy(
            tr_v.at[p, pl.ds(r * 8, 8), :],
            out_hbm.at[f, r, ctile0 + cg], osems[p])
        for r in range(SUB)
    ]

  def transpose_block(st, p):
    f = st % N_CAT
    cg = st // N_CAT
    psplat = jnp.full((LANES,), p, jnp.int32)

    def g_body(g, carry):
      r16 = iota16 + g * LANES
      idx16 = idx_v[f, pl.ds(cg * GC + g * LANES, LANES)]
      cb16 = (idx16 & 3) * DIM
      tmask = idx16 >= (NROW - TAIL)
      tloc = jnp.maximum(idx16 - (NROW - TAIL), 0)
      for k in range(LANES):
        dk16 = (iota16 + k) & (LANES - 1)
        for d0 in (0, LANES):
          d16 = dk16 + d0
          vals = plsc.load_gather(gat_v, [psplat, r16, cb16 + d16])
          tvals = plsc.load_gather(tail_v, [tloc, d16])
          plsc.store_scatter(
              tr_v.at[p], [d16, r16], jnp.where(tmask, tvals, vals))
      return carry
    lax.fori_loop(0, GC // LANES, g_body, 0)

  gather_copy(0, 0).start()

  def cat_body(sto, carry):
    for p in (0, 1):
      st = sto * 2 + p

      @pl.when(st + 1 < NST)
      def _():
        gather_copy(st + 1, 1 - p).start()

      @pl.when(st >= 2)
      def _():
        for cp in out_copies(st - 2, p):
          cp.wait()
      gather_copy(st, p).wait()
      transpose_block(st, p)
      for cp in out_copies(st, p):
        cp.start()
    return carry
  lax.fori_loop(0, NST // 2, cat_body, 0)

  for p in (0, 1):
    for cp in out_copies(NST - 2 + p, p):
      cp.wait()

  def cont_copies(st, p):
    j = st % N_CONT
    cg = st // N_CONT
    return [
        pltpu.make_async_copy(
            cont_v.at[p, pl.ds(r * 8, 8), :],
            out_hbm.at[N_CAT + j, r, ctile0 + cg], csems[p])
        for r in range(SUB)
    ]

  def cont_body(sto, carry):
    for p in (0, 1):
      st = sto * 2 + p
      j = st % N_CONT
      cg = st // N_CONT

      @pl.when(st >= 2)
      def _():
        for cp in cont_copies(st - 2, p):
          cp.wait()

      def d_body(d, carry2):
        dd = d // LANES
        wv = _splat(w_v[j, pl.ds(dd * LANES, LANES)], d % LANES)
        bv = _splat(bias_v[j, pl.ds(dd * LANES, LANES)], d % LANES)
        for k in range(GC // LANES):
          sl = pl.ds(k * LANES, LANES)
          cont_v[p, d, sl] = xc_v[j, pl.ds(cg * GC + k * LANES, LANES)] * wv + bv
        return carry2
      lax.fori_loop(0, DIM, d_body, 0)
      for cp in cont_copies(st, p):
        cp.start()
    return carry
  lax.fori_loop(0, (N_CONT * NG) // 2, cont_body, 0)

  for p in (0, 1):
    for cp in cont_copies(N_CONT * NG - 2 + p, p):
      cp.wait()


_sc_kernel = functools.partial(
    pl.kernel,
    mesh=plsc.VectorSubcoreMesh(core_axis_name="c", subcore_axis_name="s"),
    compiler_params=pltpu.CompilerParams(needs_layout_passes=False),
    out_type=jax.ShapeDtypeStruct((N_TOK, SUB, B // GC, 8, GC), jnp.float32),
    scratch_types=[
        pltpu.VMEM((N_CAT, BW), jnp.int32),
        pltpu.VMEM((N_CAT, BW), jnp.int32),
        pltpu.VMEM((2, GC, 128), jnp.float32),
        pltpu.VMEM((2, DIM, GC), jnp.float32),
        pltpu.VMEM((N_CONT, BW), jnp.float32),
        pltpu.VMEM((N_CONT, DIM), jnp.float32),
        pltpu.VMEM((N_CONT, DIM), jnp.float32),
        pltpu.VMEM((TAIL, DIM), jnp.float32),
        pltpu.VMEM((2, DIM, GC), jnp.float32),
        pltpu.SemaphoreType.DMA,
        pltpu.SemaphoreType.DMA,
        pltpu.SemaphoreType.DMA,
        pltpu.SemaphoreType.DMA,
        pltpu.SemaphoreType.DMA,
        pltpu.SemaphoreType.DMA,
    ],
)(_sc_body)


@jax.jit
def kernel(x_cat, x_cont, cat_table, cont_W, cont_b):
  xcat_t = jnp.transpose(x_cat.astype(jnp.int32))
  xcont_t = jnp.transpose(x_cont)
  tt = jnp.transpose(cat_table)
  wide = _tr_kernel(tt)
  tail = cat_table[NROW - TAIL:, :]
  out5 = _sc_kernel(xcat_t, xcont_t, wide, cont_W, cont_b, tail)
  out_t = jnp.transpose(out5, (0, 1, 3, 2, 4)).reshape(N_TOK, DIM, B)
  return jnp.transpose(out_t, (2, 0, 1))


# speedup vs baseline: 2.3674x; 1.0109x over previous
import functools

import jax
import jax.numpy as jnp
from jax import lax
from jax.experimental import pallas as pl
from jax.experimental.pallas import tpu as pltpu
from jax.experimental.pallas import tpu_sc as plsc

B = 16384
N_CAT = 26
N_CONT = 16
CARD = 100000
DIM = 32
N_TOK = N_CAT + N_CONT
NROW = N_CAT * CARD
PACK = 128 // DIM
NWIDE = NROW // PACK

NC = 2
NS = 16
NW = NC * NS
BW = B // NW
GC = 128
NG = BW // GC
NST = N_CAT * NG
SUB = DIM // 8
LANES = 16

NTILE = NROW // 128
TAIL = NROW - NTILE * 128
CT = 2
NCHK = NTILE // CT
CPW = NCHK // NW
CEXTRA = NCHK - CPW * NW
NBANK = 4


def _splat(vec, i):
  return lax.gather(
      vec, jnp.full((LANES, 1), i, jnp.int32),
      dimension_numbers=lax.GatherDimensionNumbers(
          offset_dims=(), collapsed_slice_dims=(0,), start_index_map=(0,)),
      slice_sizes=(1,),
      mode=lax.GatherScatterMode.PROMISE_IN_BOUNDS)


def _tr_body(tt_hbm, wide_hbm, src_v, tr_v,
             lsem0, lsem1, lsem2, lsem3, osem0, osem1, osem2, osem3):
  c = lax.axis_index("c")
  s = lax.axis_index("s")
  wid = s * NC + c
  nc = CPW + jnp.where(wid < CEXTRA, 1, 0)
  c0 = wid * CPW + jnp.minimum(wid, CEXTRA)
  lsems = (lsem0, lsem1, lsem2, lsem3)
  osems = (osem0, osem1, osem2, osem3)
  iota16 = lax.iota(jnp.int32, LANES)

  def load_copy(t, b):
    return pltpu.make_async_copy(
        tt_hbm.at[:, pl.ds((c0 + t) * (CT * 128), CT * 128)],
        src_v.at[b], lsems[b])

  def out_copy(t, b):
    return pltpu.make_async_copy(
        tr_v.at[b],
        wide_hbm.at[pl.ds((c0 + t) * (CT * DIM), CT * DIM), :], osems[b])

  for b in range(NBANK - 1):
    @pl.when(b < nc)
    def _():
      load_copy(b, b).start()

  def body(to, carry):
    for b in range(NBANK):
      t = to * NBANK + b

      @pl.when(t < nc)
      def _():
        @pl.when(t + (NBANK - 1) < nc)
        def _():
          load_copy(t + (NBANK - 1), (b + NBANK - 1) % NBANK).start()

        @pl.when(t >= NBANK)
        def _():
          out_copy(t - NBANK, b).wait()
        load_copy(t, b).wait()
        colb16 = (iota16 & 3) * DIM
        rowb16 = lax.shift_right_logical(iota16, 2)
        bsplat = jnp.full((LANES,), b, jnp.int32)

        def g_body(g, carry2):
          r16 = iota16 + g * LANES
          row16 = rowb16 + g * 4
          for k in range(LANES):
            dk16 = (iota16 + k) & (LANES - 1)
            for d0 in (0, LANES):
              d16 = dk16 + d0
              vals = plsc.load_gather(src_v, [bsplat, d16, r16])
              plsc.store_scatter(tr_v.at[b], [row16, colb16 + d16], vals)
          return carry2
        lax.fori_loop(0, (CT * 128) // LANES, g_body, 0)
        out_copy(t, b).start()
    return carry
  lax.fori_loop(0, (CPW + NBANK) // NBANK + 1, body, 0)

  for b in range(NBANK):
    out_copy(0, b).wait()


_tr_kernel = functools.partial(
    pl.kernel,
    mesh=plsc.VectorSubcoreMesh(core_axis_name="c", subcore_axis_name="s"),
    compiler_params=pltpu.CompilerParams(needs_layout_passes=False),
    out_type=jax.ShapeDtypeStruct((NWIDE, 128), jnp.float32),
    scratch_types=[
        pltpu.VMEM((NBANK, DIM, CT * 128), jnp.float32),
        pltpu.VMEM((NBANK, CT * DIM, 128), jnp.float32),
        pltpu.SemaphoreType.DMA,
        pltpu.SemaphoreType.DMA,
        pltpu.SemaphoreType.DMA,
        pltpu.SemaphoreType.DMA,
        pltpu.SemaphoreType.DMA,
        pltpu.SemaphoreType.DMA,
        pltpu.SemaphoreType.DMA,
        pltpu.SemaphoreType.DMA,
    ],
)(_tr_body)


def _sc_body(xcat_hbm, xcont_hbm, wide_hbm, w_hbm, bias_hbm, tail_hbm,
             out_hbm, idx_v, gat_v, tr_v, xc_v, w_v, bias_v,
             tail_v, cont_v, gsem0, gsem1, osem0, osem1, csem0, csem1):
  c = lax.axis_index("c")
  s = lax.axis_index("s")
  wid = s * NC + c
  base = wid * BW
  ctile0 = wid * NG
  gsems = (gsem0, gsem1)
  osems = (osem0, osem1)
  csems = (csem0, csem1)
  iota16 = lax.iota(jnp.int32, LANES)

  pltpu.sync_copy(w_hbm, w_v)
  pltpu.sync_copy(bias_hbm, bias_v)
  pltpu.sync_copy(tail_hbm, tail_v)
  pltpu.sync_copy(xcont_hbm.at[:, pl.ds(base, BW)], xc_v)
  pltpu.sync_copy(xcat_hbm.at[:, pl.ds(base, BW)], idx_v)

  def add_body(f, carry):
    off = f * CARD
    for k in range(BW // LANES):
      sl = pl.ds(k * LANES, LANES)
      idx_v[f, sl] = idx_v[f, sl] + off
    return carry
  lax.fori_loop(0, N_CAT, add_body, 0)

  def gather_copy(st, p):
    f = st % N_CAT
    cg = st // N_CAT
    return pltpu.make_async_copy(
        wide_hbm.at[idx_v.at[f, pl.ds(cg * GC, GC)]],
        gat_v.at[p], gsems[p])

  def out_copies(st, p):
    f = st % N_CAT
    cg = st // N_CAT
    return [
        pltpu.make_async_copy(
            tr_v.at[p, pl.ds(r * 8, 8), :],
            out_hbm.at[f, r, ctile0 + cg], osems[p])
        for r in range(SUB)
    ]

  def transpose_block(st, p):
    f = st % N_CAT
    cg = st // N_CAT
    psplat = jnp.full((LANES,), p, jnp.int32)

    def g_body(g, carry):
      r16 = iota16 + g * LANES
      idx16 = idx_v[f, pl.ds(cg * GC + g * LANES, LANES)]
      tmask = idx16 >= (NROW - TAIL)
      tloc = jnp.maximum(idx16 - (NROW - TAIL), 0)
      for k in range(LANES):
        dk16 = (iota16 + k) & (LANES - 1)
        for d0 in (0, LANES):
          d16 = dk16 + d0
          vals = plsc.load_gather(gat_v, [psplat, r16, d16])
          tvals = plsc.load_gather(tail_v, [tloc, d16])
          plsc.store_scatter(
              tr_v.at[p], [d16, r16], jnp.where(tmask, tvals, vals))
      return carry
    lax.fori_loop(0, GC // LANES, g_body, 0)

  gather_copy(0, 0).start()

  def cat_body(sto, carry):
    for p in (0, 1):
      st = sto * 2 + p

      @pl.when(st + 1 < NST)
      def _():
        gather_copy(st + 1, 1 - p).start()

      @pl.when(st >= 2)
      def _():
        for cp in out_copies(st - 2, p):
          cp.wait()
      gather_copy(st, p).wait()
      transpose_block(st, p)
      for cp in out_copies(st, p):
        cp.start()
    return carry
  lax.fori_loop(0, NST // 2, cat_body, 0)

  for p in (0, 1):
    for cp in out_copies(NST - 2 + p, p):
      cp.wait()

  def cont_copies(st, p):
    j = st % N_CONT
    cg = st // N_CONT
    return [
        pltpu.make_async_copy(
            cont_v.at[p, pl.ds(r * 8, 8), :],
            out_hbm.at[N_CAT + j, r, ctile0 + cg], csems[p])
        for r in range(SUB)
    ]

  def cont_body(sto, carry):
    for p in (0, 1):
      st = sto * 2 + p
      j = st % N_CONT
      cg = st // N_CONT

      @pl.when(st >= 2)
      def _():
        for cp in cont_copies(st - 2, p):
          cp.wait()

      def d_body(d, carry2):
        dd = d // LANES
        wv = _splat(w_v[j, pl.ds(dd * LANES, LANES)], d % LANES)
        bv = _splat(bias_v[j, pl.ds(dd * LANES, LANES)], d % LANES)
        for k in range(GC // LANES):
          sl = pl.ds(k * LANES, LANES)
          cont_v[p, d, sl] = xc_v[j, pl.ds(cg * GC + k * LANES, LANES)] * wv + bv
        return carry2
      lax.fori_loop(0, DIM, d_body, 0)
      for cp in cont_copies(st, p):
        cp.start()
    return carry
  lax.fori_loop(0, (N_CONT * NG) // 2, cont_body, 0)

  for p in (0, 1):
    for cp in cont_copies(N_CONT * NG - 2 + p, p):
      cp.wait()


_sc_kernel = functools.partial(
    pl.kernel,
    mesh=plsc.VectorSubcoreMesh(core_axis_name="c", subcore_axis_name="s"),
    compiler_params=pltpu.CompilerParams(
        use_tc_tiling_on_sc=False, needs_layout_passes=False),
    out_type=jax.ShapeDtypeStruct((N_TOK, SUB, B // GC, 8, GC), jnp.float32),
    scratch_types=[
        pltpu.VMEM((N_CAT, BW), jnp.int32),
        pltpu.VMEM((2, GC, DIM), jnp.float32),
        pltpu.VMEM((2, DIM, GC), jnp.float32),
        pltpu.VMEM((N_CONT, BW), jnp.float32),
        pltpu.VMEM((N_CONT, DIM), jnp.float32),
        pltpu.VMEM((N_CONT, DIM), jnp.float32),
        pltpu.VMEM((TAIL, DIM), jnp.float32),
        pltpu.VMEM((2, DIM, GC), jnp.float32),
        pltpu.SemaphoreType.DMA,
        pltpu.SemaphoreType.DMA,
        pltpu.SemaphoreType.DMA,
        pltpu.SemaphoreType.DMA,
        pltpu.SemaphoreType.DMA,
        pltpu.SemaphoreType.DMA,
    ],
)(_sc_body)


@jax.jit
def kernel(x_cat, x_cont, cat_table, cont_W, cont_b):
  xcat_t = jnp.transpose(x_cat.astype(jnp.int32))
  xcont_t = jnp.transpose(x_cont)
  tt = jnp.transpose(cat_table)
  wide = _tr_kernel(tt)
  tail = cat_table[NROW - TAIL:, :]
  out5 = _sc_kernel(xcat_t, xcont_t, wide.reshape(NROW, DIM), cont_W,
                    cont_b, tail)
  out_t = jnp.transpose(out5, (0, 1, 3, 2, 4)).reshape(N_TOK, DIM, B)
  return jnp.transpose(out_t, (2, 0, 1))


# speedup vs baseline: 2.4629x; 1.0404x over previous
import functools

import jax
import jax.numpy as jnp
from jax import lax
from jax.experimental import pallas as pl
from jax.experimental.pallas import tpu as pltpu
from jax.experimental.pallas import tpu_sc as plsc

B = 16384
N_CAT = 26
N_CONT = 16
CARD = 100000
DIM = 32
N_TOK = N_CAT + N_CONT
NROW = N_CAT * CARD
PACK = 128 // DIM
NWIDE = NROW // PACK

NC = 2
NS = 16
NW = NC * NS
BW = B // NW
GC = 128
NG = BW // GC
NST = N_CAT * NG
SUB = DIM // 8
LANES = 16

NTILE = NROW // 128
TAIL = NROW - NTILE * 128
CT = 4
NCHK = NTILE // CT
CPW = NCHK // NW
CEXTRA = NCHK - CPW * NW
NBANK = 3


def _splat(vec, i):
  return lax.gather(
      vec, jnp.full((LANES, 1), i, jnp.int32),
      dimension_numbers=lax.GatherDimensionNumbers(
          offset_dims=(), collapsed_slice_dims=(0,), start_index_map=(0,)),
      slice_sizes=(1,),
      mode=lax.GatherScatterMode.PROMISE_IN_BOUNDS)


def _tr_body(tt_hbm, wide_hbm, src_v, tr_v,
             lsem0, lsem1, lsem2, osem0, osem1, osem2):
  c = lax.axis_index("c")
  s = lax.axis_index("s")
  wid = s * NC + c
  nc = CPW + jnp.where(wid < CEXTRA, 1, 0)
  c0 = wid * CPW + jnp.minimum(wid, CEXTRA)
  lsems = (lsem0, lsem1, lsem2)
  osems = (osem0, osem1, osem2)
  iota16 = lax.iota(jnp.int32, LANES)

  def load_copy(t, b):
    return pltpu.make_async_copy(
        tt_hbm.at[:, pl.ds((c0 + t) * (CT * 128), CT * 128)],
        src_v.at[b], lsems[b])

  def out_copy(t, b):
    return pltpu.make_async_copy(
        tr_v.at[b],
        wide_hbm.at[pl.ds((c0 + t) * (CT * DIM), CT * DIM), :], osems[b])

  for b in range(NBANK - 1):
    @pl.when(b < nc)
    def _():
      load_copy(b, b).start()

  def body(to, carry):
    for b in range(NBANK):
      t = to * NBANK + b

      @pl.when(t < nc)
      def _():
        @pl.when(t + (NBANK - 1) < nc)
        def _():
          load_copy(t + (NBANK - 1), (b + NBANK - 1) % NBANK).start()

        @pl.when(t >= NBANK)
        def _():
          out_copy(t - NBANK, b).wait()
        load_copy(t, b).wait()
        colb16 = (iota16 & 3) * DIM
        rowb16 = lax.shift_right_logical(iota16, 2)
        bsplat = jnp.full((LANES,), b, jnp.int32)
        dks = [(iota16 + k) & (LANES - 1) for k in range(LANES)]

        def g_body(g, carry2):
          r16 = iota16 + g * LANES
          row16 = rowb16 + g * 4
          for k in range(LANES):
            for d0 in (0, LANES):
              d16 = dks[k] + d0
              vals = plsc.load_gather(src_v, [bsplat, d16, r16])
              plsc.store_scatter(tr_v.at[b], [row16, colb16 + d16], vals)
          return carry2
        lax.fori_loop(0, (CT * 128) // LANES, g_body, 0)
        out_copy(t, b).start()
    return carry
  lax.fori_loop(0, (CPW + NBANK) // NBANK + 1, body, 0)

  for b in range(NBANK):
    out_copy(0, b).wait()


_tr_kernel = functools.partial(
    pl.kernel,
    mesh=plsc.VectorSubcoreMesh(core_axis_name="c", subcore_axis_name="s"),
    compiler_params=pltpu.CompilerParams(needs_layout_passes=False),
    out_type=jax.ShapeDtypeStruct((NWIDE, 128), jnp.float32),
    scratch_types=[
        pltpu.VMEM((NBANK, DIM, CT * 128), jnp.float32),
        pltpu.VMEM((NBANK, CT * DIM, 128), jnp.float32),
        pltpu.SemaphoreType.DMA,
        pltpu.SemaphoreType.DMA,
        pltpu.SemaphoreType.DMA,
        pltpu.SemaphoreType.DMA,
        pltpu.SemaphoreType.DMA,
        pltpu.SemaphoreType.DMA,
    ],
)(_tr_body)


def _sc_body(xcat_hbm, xcont_hbm, wide_hbm, w_hbm, bias_hbm, tail_hbm,
             out_hbm, idx_v, gat_v, tr_v, xc_v, w_v, bias_v,
             tail_v, cont_v, gsem0, gsem1, osem0, osem1, csem0, csem1):
  c = lax.axis_index("c")
  s = lax.axis_index("s")
  wid = s * NC + c
  base = wid * BW
  ctile0 = wid * NG
  gsems = (gsem0, gsem1)
  osems = (osem0, osem1)
  csems = (csem0, csem1)
  iota16 = lax.iota(jnp.int32, LANES)

  pltpu.sync_copy(w_hbm, w_v)
  pltpu.sync_copy(bias_hbm, bias_v)
  pltpu.sync_copy(tail_hbm, tail_v)
  pltpu.sync_copy(xcont_hbm.at[:, pl.ds(base, BW)], xc_v)
  pltpu.sync_copy(xcat_hbm.at[:, pl.ds(base, BW)], idx_v)

  def add_body(f, carry):
    off = f * CARD
    for k in range(BW // LANES):
      sl = pl.ds(k * LANES, LANES)
      idx_v[f, sl] = idx_v[f, sl] + off
    return carry
  lax.fori_loop(0, N_CAT, add_body, 0)

  def gather_copy(st, p):
    f = st % N_CAT
    cg = st // N_CAT
    return pltpu.make_async_copy(
        wide_hbm.at[idx_v.at[f, pl.ds(cg * GC, GC)]],
        gat_v.at[p], gsems[p])

  def out_copies(st, p):
    f = st % N_CAT
    cg = st // N_CAT
    return [
        pltpu.make_async_copy(
            tr_v.at[p, pl.ds(r * 8, 8), :],
            out_hbm.at[f, r, ctile0 + cg], osems[p])
        for r in range(SUB)
    ]

  def transpose_block(st, p):
    f = st % N_CAT
    cg = st // N_CAT
    psplat = jnp.full((LANES,), p, jnp.int32)

    def g_body(g, carry):
      r16 = iota16 + g * LANES
      idx16 = idx_v[f, pl.ds(cg * GC + g * LANES, LANES)]
      tmask = idx16 >= (NROW - TAIL)
      tloc = jnp.maximum(idx16 - (NROW - TAIL), 0)
      for k in range(LANES):
        dk16 = (iota16 + k) & (LANES - 1)
        for d0 in (0, LANES):
          d16 = dk16 + d0
          vals = plsc.load_gather(gat_v, [psplat, r16, d16])
          tvals = plsc.load_gather(tail_v, [tloc, d16])
          plsc.store_scatter(
              tr_v.at[p], [d16, r16], jnp.where(tmask, tvals, vals))
      return carry
    lax.fori_loop(0, GC // LANES, g_body, 0)

  gather_copy(0, 0).start()

  def cat_body(sto, carry):
    for p in (0, 1):
      st = sto * 2 + p

      @pl.when(st + 1 < NST)
      def _():
        gather_copy(st + 1, 1 - p).start()

      @pl.when(st >= 2)
      def _():
        for cp in out_copies(st - 2, p):
          cp.wait()
      gather_copy(st, p).wait()
      transpose_block(st, p)
      for cp in out_copies(st, p):
        cp.start()
    return carry
  lax.fori_loop(0, NST // 2, cat_body, 0)

  for p in (0, 1):
    for cp in out_copies(NST - 2 + p, p):
      cp.wait()

  def cont_copies(st, p):
    j = st % N_CONT
    cg = st // N_CONT
    return [
        pltpu.make_async_copy(
            cont_v.at[p, pl.ds(r * 8, 8), :],
            out_hbm.at[N_CAT + j, r, ctile0 + cg], csems[p])
        for r in range(SUB)
    ]

  def cont_body(sto, carry):
    for p in (0, 1):
      st = sto * 2 + p
      j = st % N_CONT
      cg = st // N_CONT

      @pl.when(st >= 2)
      def _():
        for cp in cont_copies(st - 2, p):
          cp.wait()

      def d_body(d, carry2):
        dd = d // LANES
        wv = _splat(w_v[j, pl.ds(dd * LANES, LANES)], d % LANES)
        bv = _splat(bias_v[j, pl.ds(dd * LANES, LANES)], d % LANES)
        for k in range(GC // LANES):
          sl = pl.ds(k * LANES, LANES)
          cont_v[p, d, sl] = xc_v[j, pl.ds(cg * GC + k * LANES, LANES)] * wv + bv
        return carry2
      lax.fori_loop(0, DIM, d_body, 0)
      for cp in cont_copies(st, p):
        cp.start()
    return carry
  lax.fori_loop(0, (N_CONT * NG) // 2, cont_body, 0)

  for p in (0, 1):
    for cp in cont_copies(N_CONT * NG - 2 + p, p):
      cp.wait()


_sc_kernel = functools.partial(
    pl.kernel,
    mesh=plsc.VectorSubcoreMesh(core_axis_name="c", subcore_axis_name="s"),
    compiler_params=pltpu.CompilerParams(
        use_tc_tiling_on_sc=False, needs_layout_passes=False),
    out_type=jax.ShapeDtypeStruct((N_TOK, SUB, B // GC, 8, GC), jnp.float32),
    scratch_types=[
        pltpu.VMEM((N_CAT, BW), jnp.int32),
        pltpu.VMEM((2, GC, DIM), jnp.float32),
        pltpu.VMEM((2, DIM, GC), jnp.float32),
        pltpu.VMEM((N_CONT, BW), jnp.float32),
        pltpu.VMEM((N_CONT, DIM), jnp.float32),
        pltpu.VMEM((N_CONT, DIM), jnp.float32),
        pltpu.VMEM((TAIL, DIM), jnp.float32),
        pltpu.VMEM((2, DIM, GC), jnp.float32),
        pltpu.SemaphoreType.DMA,
        pltpu.SemaphoreType.DMA,
        pltpu.SemaphoreType.DMA,
        pltpu.SemaphoreType.DMA,
        pltpu.SemaphoreType.DMA,
        pltpu.SemaphoreType.DMA,
    ],
)(_sc_body)


@jax.jit
def kernel(x_cat, x_cont, cat_table, cont_W, cont_b):
  xcat_t = jnp.transpose(x_cat.astype(jnp.int32))
  xcont_t = jnp.transpose(x_cont)
  tt = jnp.transpose(cat_table)
  wide = _tr_kernel(tt)
  tail = cat_table[NROW - TAIL:, :]
  out5 = _sc_kernel(xcat_t, xcont_t, wide.reshape(NROW, DIM), cont_W,
                    cont_b, tail)
  out_t = jnp.transpose(out5, (0, 1, 3, 2, 4)).reshape(N_TOK, DIM, B)
  return jnp.transpose(out_t, (2, 0, 1))
